# baseline (speedup 1.0000x reference)
import functools

import jax
import jax.numpy as jnp
from jax.experimental import pallas as pl
from jax.experimental.pallas import tpu as pltpu

_H = 8
_NB = 5
_RB = 256


def _attn_rows(x_ref, wq_ref, wk_ref, kt, r, scale):
    xr = x_ref[0, r * _RB:(r + 1) * _RB, :]
    q = jax.lax.dot_general(xr, wq_ref[...],
                            (((1,), (1,)), ((), ())),
                            preferred_element_type=jnp.float32)
    logits = jax.lax.dot_general(q, kt,
                                 (((1,), (0,)), ((), ())),
                                 preferred_element_type=jnp.float32) * scale
    m = jnp.max(logits, axis=-1, keepdims=True)
    e = jnp.exp(logits - m)
    return e / jnp.sum(e, axis=-1, keepdims=True)


def _make_kt(x_ref, wk_ref):
    return jax.lax.dot_general(wk_ref[...], x_ref[0],
                               (((1,), (1,)), ((), ())),
                               preferred_element_type=jnp.float32)


def _colmask_kernel(x_ref, wq_ref, wk_ref, cm_ref, se_ref):
    h = pl.program_id(1)
    N = se_ref.shape[-1]
    C = x_ref.shape[-1]
    scale = C ** -0.5
    kt = _make_kt(x_ref, wk_ref)
    for r in range(N // _RB):
        attn = _attn_rows(x_ref, wq_ref, wk_ref, kt, r, scale)
        sl = pl.ds(r * _RB, _RB)

        @pl.when(h == 0)
        def _():
            se_ref[sl, :] = attn

        @pl.when(h > 0)
        def _():
            se_ref[sl, :] = se_ref[sl, :] + attn

    @pl.when(h == _H - 1)
    def _():
        cm = jnp.zeros((1, N), jnp.float32)
        for r in range(N // _RB):
            se = se_ref[r * _RB:(r + 1) * _RB, :]
            col = jax.lax.broadcasted_iota(jnp.int32, (_RB, N), 1)
            for _ in range(_NB):
                idx = jnp.argmax(se, axis=-1, keepdims=True)
                se = jnp.where(col == idx, -1.0, se)
            hit = jnp.max(jnp.where(se == -1.0, 1.0, 0.0),
                          axis=0, keepdims=True)
            cm = jnp.maximum(cm, hit)
        cm_ref[...] = cm.reshape(1, 1, N)


def _edge_kernel(x_ref, wq_ref, wk_ref, cm_ref, out_ref, nr_ref):
    N = nr_ref.shape[-1]
    C = x_ref.shape[-1]
    scale = C ** -0.5
    cm = cm_ref[0]
    kt = _make_kt(x_ref, wk_ref)
    s = jnp.zeros((1, N), jnp.float32)
    for r in range(N // _RB):
        attn = _attn_rows(x_ref, wq_ref, wk_ref, kt, r, scale)
        col = jax.lax.broadcasted_iota(jnp.int32, (_RB, N), 1)
        row = jax.lax.broadcasted_iota(jnp.int32, (_RB, N), 0) + r * _RB
        ne = jnp.where(col == row, attn, attn * cm)
        rs = jnp.sum(ne, axis=-1, keepdims=True) + 1e-16
        nr = ne / rs
        nr_ref[r * _RB:(r + 1) * _RB, :] = nr.astype(jnp.bfloat16)
        s = s + jnp.sum(nr, axis=0, keepdims=True)
    sinv = (1.0 / (s + 1e-16)).astype(jnp.bfloat16)
    for r in range(N // _RB):
        a = nr_ref[r * _RB:(r + 1) * _RB, :] * sinv
        out_ref[0, 0, r * _RB:(r + 1) * _RB, :] = jax.lax.dot_general(
            a, nr_ref[...], (((1,), (1,)), ((), ())),
            preferred_element_type=jnp.float32)


def _run(x, W_qk):
    B, N, C = x.shape
    f32 = jnp.float32

    colmask = pl.pallas_call(
        _colmask_kernel,
        grid=(B, _H),
        in_specs=[
            pl.BlockSpec((1, N, C), lambda b, h: (b, 0, 0)),
            pl.BlockSpec((C, C), lambda b, h: (h, 0)),
            pl.BlockSpec((C, C), lambda b, h: (_H + h, 0)),
        ],
        out_specs=pl.BlockSpec((1, 1, N), lambda b, h: (b, 0, 0)),
        out_shape=jax.ShapeDtypeStruct((B, 1, N), f32),
        scratch_shapes=[pltpu.VMEM((N, N), f32)],
        compiler_params=pltpu.CompilerParams(
            dimension_semantics=("parallel", "arbitrary")),
    )(x, W_qk, W_qk)

    out = pl.pallas_call(
        _edge_kernel,
        grid=(B, _H),
        in_specs=[
            pl.BlockSpec((1, N, C), lambda b, h: (b, 0, 0)),
            pl.BlockSpec((C, C), lambda b, h: (h, 0)),
            pl.BlockSpec((C, C), lambda b, h: (_H + h, 0)),
            pl.BlockSpec((1, 1, N), lambda b, h: (b, 0, 0)),
        ],
        out_specs=pl.BlockSpec((1, 1, N, N), lambda b, h: (b, h, 0, 0)),
        out_shape=jax.ShapeDtypeStruct((B, _H, N, N), f32),
        scratch_shapes=[pltpu.VMEM((N, N), jnp.bfloat16)],
        compiler_params=pltpu.CompilerParams(
            dimension_semantics=("parallel", "arbitrary")),
    )(x, W_qk, W_qk, colmask)
    return out


@jax.jit
def kernel(x, W_qk):
    devs = jax.devices()
    if len(devs) >= 2 and x.shape[0] % 2 == 0:
        mesh = jax.sharding.Mesh(devs[:2], ("d",))
        spec = jax.sharding.PartitionSpec
        return jax.shard_map(
            _run, mesh=mesh,
            in_specs=(spec("d"), spec()),
            out_specs=spec("d"), check_vma=False)(x, W_qk)
    return _run(x, W_qk)


# speedup vs baseline: 1.4839x; 1.4839x over previous
import jax
import jax.numpy as jnp
from jax.experimental import pallas as pl
from jax.experimental.pallas import tpu as pltpu

_H = 8
_NB = 5
_RB = 256
_TB = 64


def _fused_kernel(x_ref, wq_ref, wk_ref, out_ref, stash_ref, se_ref, cm_ref):
    p = pl.program_id(1)
    h = pl.program_id(2)
    N = se_ref.shape[-1]
    C = x_ref.shape[-1]
    scale = C ** -0.5

    @pl.when(p == 0)
    def _phase0():
        kt = jax.lax.dot_general(wk_ref[...], x_ref[0],
                                 (((1,), (1,)), ((), ())),
                                 preferred_element_type=jnp.float32)
        for r in range(N // _RB):
            sl = pl.ds(r * _RB, _RB)
            xr = x_ref[0, sl, :]
            q = jax.lax.dot_general(xr, wq_ref[...],
                                    (((1,), (1,)), ((), ())),
                                    preferred_element_type=jnp.float32)
            logits = jax.lax.dot_general(q, kt,
                                         (((1,), (0,)), ((), ())),
                                         preferred_element_type=jnp.float32)
            logits = logits * scale
            m = jnp.max(logits, axis=-1, keepdims=True)
            e = jnp.exp(logits - m)
            stash_ref[h, sl, :] = e.astype(jnp.bfloat16)
            attn = e / jnp.sum(e, axis=-1, keepdims=True)

            @pl.when(h == 0)
            def _():
                se_ref[sl, :] = attn

            @pl.when(h > 0)
            def _():
                se_ref[sl, :] = se_ref[sl, :] + attn

        @pl.when(h == _H - 1)
        def _():
            cm = jnp.zeros((1, N), jnp.float32)
            for r in range(N // _TB):
                se = se_ref[r * _TB:(r + 1) * _TB, :]
                col = jax.lax.broadcasted_iota(jnp.int32, (_TB, N), 1)
                for _ in range(_NB):
                    v = jnp.max(se, axis=-1, keepdims=True)
                    first = jnp.min(jnp.where(se == v, col, N),
                                    axis=-1, keepdims=True)
                    se = jnp.where(col == first, -1.0, se)
                hit = jnp.max(jnp.where(se == -1.0, 1.0, 0.0),
                              axis=0, keepdims=True)
                cm = jnp.maximum(cm, hit)
            cm_ref[...] = cm

    @pl.when(p == 1)
    def _phase1():
        cm = cm_ref[...]
        row0 = jax.lax.broadcasted_iota(jnp.int32, (_RB, N), 0)
        col = jax.lax.broadcasted_iota(jnp.int32, (_RB, N), 1)
        s = jnp.zeros((1, N), jnp.float32)
        for r in range(N // _RB):
            sl = pl.ds(r * _RB, _RB)
            e = stash_ref[h, sl, :].astype(jnp.float32)
            ne = jnp.where(col == row0 + r * _RB, e, e * cm)
            rs = jnp.sum(ne, axis=-1, keepdims=True) + 1e-16
            nr = ne / rs
            stash_ref[h, sl, :] = nr.astype(jnp.bfloat16)
            s = s + jnp.sum(nr, axis=0, keepdims=True)
        sinv = (1.0 / (s + 1e-16)).astype(jnp.bfloat16)
        for r in range(N // _RB):
            a = stash_ref[h, r * _RB:(r + 1) * _RB, :] * sinv
            out_ref[0, 0, r * _RB:(r + 1) * _RB, :] = jax.lax.dot_general(
                a, stash_ref[h], (((1,), (1,)), ((), ())),
                preferred_element_type=jnp.float32)


@jax.jit
def kernel(x, W_qk):
    B, N, C = x.shape
    f32 = jnp.float32
    out = pl.pallas_call(
        _fused_kernel,
        grid=(B, 2, _H),
        in_specs=[
            pl.BlockSpec((1, N, C), lambda b, p, h: (b, 0, 0)),
            pl.BlockSpec((C, C), lambda b, p, h: (h, 0)),
            pl.BlockSpec((C, C), lambda b, p, h: (_H + h, 0)),
        ],
        out_specs=pl.BlockSpec((1, 1, N, N), lambda b, p, h: (b, p * h, 0, 0)),
        out_shape=jax.ShapeDtypeStruct((B, _H, N, N), f32),
        scratch_shapes=[
            pltpu.VMEM((_H, N, N), jnp.bfloat16),
            pltpu.VMEM((N, N), f32),
            pltpu.VMEM((1, N), f32),
        ],
        compiler_params=pltpu.CompilerParams(
            dimension_semantics=("arbitrary", "arbitrary", "arbitrary"),
            vmem_limit_bytes=100 * 1024 * 1024),
    )(x, W_qk, W_qk)
    return out


# speedup vs baseline: 1.4985x; 1.0099x over previous
import jax
import jax.numpy as jnp
from jax.experimental import pallas as pl
from jax.experimental.pallas import tpu as pltpu

_H = 8
_NB = 5
_RB = 256
_TB = 64


def _fused_kernel(x_ref, wq_ref, wk_ref, out_ref, stash_ref, se_ref, cm_ref):
    p = pl.program_id(1)
    h = pl.program_id(2)
    N = se_ref.shape[-1]
    C = x_ref.shape[-1]
    scale = C ** -0.5

    @pl.when(p == 0)
    def _phase0():
        kt = jax.lax.dot_general(wk_ref[...], x_ref[0],
                                 (((1,), (1,)), ((), ())),
                                 preferred_element_type=jnp.float32)
        log2e = 1.4426950408889634
        for r in range(N // _RB):
            sl = pl.ds(r * _RB, _RB)
            xr = x_ref[0, sl, :]
            q = jax.lax.dot_general(xr, wq_ref[...],
                                    (((1,), (1,)), ((), ())),
                                    preferred_element_type=jnp.float32)
            q = q * (scale * log2e)
            logits = jax.lax.dot_general(q, kt,
                                         (((1,), (0,)), ((), ())),
                                         preferred_element_type=jnp.float32)
            m = jnp.max(logits, axis=-1, keepdims=True)
            e = jnp.exp2(logits - m)
            stash_ref[h, sl, :] = e.astype(jnp.bfloat16)
            attn = e / jnp.sum(e, axis=-1, keepdims=True)

            @pl.when(h == 0)
            def _():
                se_ref[sl, :] = attn

            @pl.when(h > 0)
            def _():
                se_ref[sl, :] = se_ref[sl, :] + attn

        @pl.when(h == _H - 1)
        def _():
            cm = jnp.zeros((1, N), jnp.float32)
            for r in range(N // _TB):
                se = se_ref[r * _TB:(r + 1) * _TB, :]
                col = jax.lax.broadcasted_iota(jnp.int32, (_TB, N), 1)
                for _ in range(_NB):
                    v = jnp.max(se, axis=-1, keepdims=True)
                    first = jnp.min(jnp.where(se == v, col, N),
                                    axis=-1, keepdims=True)
                    se = jnp.where(col == first, -1.0, se)
                hit = jnp.max(jnp.where(se == -1.0, 1.0, 0.0),
                              axis=0, keepdims=True)
                cm = jnp.maximum(cm, hit)
            cm_ref[...] = cm

    @pl.when(p == 1)
    def _phase1():
        cm = cm_ref[...]
        row0 = jax.lax.broadcasted_iota(jnp.int32, (_RB, N), 0)
        col = jax.lax.broadcasted_iota(jnp.int32, (_RB, N), 1)
        s = jnp.zeros((1, N), jnp.float32)
        for r in range(N // _RB):
            sl = pl.ds(r * _RB, _RB)
            e = stash_ref[h, sl, :].astype(jnp.float32)
            ne = e * jnp.where(col == row0 + r * _RB, 1.0, cm)
            rs = jnp.sum(ne, axis=-1, keepdims=True) + 1e-16
            nr = ne / rs
            stash_ref[h, sl, :] = nr.astype(jnp.bfloat16)
            s = s + jnp.sum(nr, axis=0, keepdims=True)
        sinv = (1.0 / (s + 1e-16)).astype(jnp.bfloat16)
        for r in range(N // _RB):
            a = stash_ref[h, r * _RB:(r + 1) * _RB, :] * sinv
            out_ref[0, 0, r * _RB:(r + 1) * _RB, :] = jax.lax.dot_general(
                a, stash_ref[h], (((1,), (1,)), ((), ())),
                preferred_element_type=jnp.float32)


@jax.jit
def kernel(x, W_qk):
    B, N, C = x.shape
    f32 = jnp.float32
    out = pl.pallas_call(
        _fused_kernel,
        grid=(B, 2, _H),
        in_specs=[
            pl.BlockSpec((1, N, C), lambda b, p, h: (b, 0, 0)),
            pl.BlockSpec((C, C), lambda b, p, h: (h, 0)),
            pl.BlockSpec((C, C), lambda b, p, h: (_H + h, 0)),
        ],
        out_specs=pl.BlockSpec((1, 1, N, N), lambda b, p, h: (b, p * h, 0, 0)),
        out_shape=jax.ShapeDtypeStruct((B, _H, N, N), f32),
        scratch_shapes=[
            pltpu.VMEM((_H, N, N), jnp.bfloat16),
            pltpu.VMEM((N, N), f32),
            pltpu.VMEM((1, N), f32),
        ],
        compiler_params=pltpu.CompilerParams(
            dimension_semantics=("arbitrary", "arbitrary", "arbitrary"),
            vmem_limit_bytes=100 * 1024 * 1024),
    )(x, W_qk, W_qk)
    return out


# speedup vs baseline: 1.6067x; 1.0722x over previous
import jax
import jax.numpy as jnp
from jax.experimental import pallas as pl
from jax.experimental.pallas import tpu as pltpu

_H = 8
_NB = 5
_RB = 256
_TB = 64


def _fused_kernel(x_ref, wq_ref, wk_ref, out_ref, stash_ref, se_ref, cm_ref):
    p = pl.program_id(1)
    h = pl.program_id(2)
    N = se_ref.shape[-1]
    C = x_ref.shape[-1]
    scale = C ** -0.5

    @pl.when(p == 0)
    def _phase0():
        kt = jax.lax.dot_general(wk_ref[...], x_ref[0],
                                 (((1,), (1,)), ((), ())),
                                 preferred_element_type=jnp.float32)
        log2e = 1.4426950408889634
        for r in range(N // _RB):
            sl = pl.ds(r * _RB, _RB)
            xr = x_ref[0, sl, :]
            q = jax.lax.dot_general(xr, wq_ref[...],
                                    (((1,), (1,)), ((), ())),
                                    preferred_element_type=jnp.float32)
            q = q * (scale * log2e)
            logits = jax.lax.dot_general(q, kt,
                                         (((1,), (0,)), ((), ())),
                                         preferred_element_type=jnp.float32)
            e = jnp.exp2(logits)
            stash_ref[h, sl, :] = e.astype(jnp.bfloat16)
            attn = e / jnp.sum(e, axis=-1, keepdims=True)

            @pl.when(h == 0)
            def _():
                se_ref[sl, :] = attn

            @pl.when(h > 0)
            def _():
                se_ref[sl, :] = se_ref[sl, :] + attn

        @pl.when(h == _H - 1)
        def _():
            cm = jnp.zeros((1, N), jnp.float32)
            for r in range(N // _TB):
                se = se_ref[r * _TB:(r + 1) * _TB, :]
                col = jax.lax.broadcasted_iota(jnp.int32, (_TB, N), 1)
                for _ in range(_NB):
                    v = jnp.max(se, axis=-1, keepdims=True)
                    first = jnp.min(jnp.where(se == v, col, N),
                                    axis=-1, keepdims=True)
                    se = jnp.where(col == first, -1.0, se)
                hit = jnp.max(jnp.where(se == -1.0, 1.0, 0.0),
                              axis=0, keepdims=True)
                cm = jnp.maximum(cm, hit)
            cm_ref[...] = cm

    @pl.when(p == 1)
    def _phase1():
        cm = cm_ref[...]
        row0 = jax.lax.broadcasted_iota(jnp.int32, (_RB, N), 0)
        col = jax.lax.broadcasted_iota(jnp.int32, (_RB, N), 1)
        s = jnp.zeros((1, N), jnp.float32)
        for r in range(N // _RB):
            sl = pl.ds(r * _RB, _RB)
            e = stash_ref[h, sl, :].astype(jnp.float32)
            ne = e * jnp.where(col == row0 + r * _RB, 1.0, cm)
            rs = jnp.sum(ne, axis=-1, keepdims=True) + 1e-16
            nr = ne / rs
            stash_ref[h, sl, :] = nr.astype(jnp.bfloat16)
            s = s + jnp.sum(nr, axis=0, keepdims=True)
        sinv = (1.0 / (s + 1e-16)).astype(jnp.bfloat16)
        for r in range(N // _RB):
            a = stash_ref[h, r * _RB:(r + 1) * _RB, :] * sinv
            out_ref[0, 0, r * _RB:(r + 1) * _RB, :] = jax.lax.dot_general(
                a, stash_ref[h], (((1,), (1,)), ((), ())),
                preferred_element_type=jnp.float32)


@jax.jit
def kernel(x, W_qk):
    B, N, C = x.shape
    f32 = jnp.float32
    out = pl.pallas_call(
        _fused_kernel,
        grid=(B, 2, _H),
        in_specs=[
            pl.BlockSpec((1, N, C), lambda b, p, h: (b, 0, 0)),
            pl.BlockSpec((C, C), lambda b, p, h: (h, 0)),
            pl.BlockSpec((C, C), lambda b, p, h: (_H + h, 0)),
        ],
        out_specs=pl.BlockSpec((1, 1, N, N), lambda b, p, h: (b, p * h, 0, 0)),
        out_shape=jax.ShapeDtypeStruct((B, _H, N, N), f32),
        scratch_shapes=[
            pltpu.VMEM((_H, N, N), jnp.bfloat16),
            pltpu.VMEM((N, N), f32),
            pltpu.VMEM((1, N), f32),
        ],
        compiler_params=pltpu.CompilerParams(
            dimension_semantics=("arbitrary", "arbitrary", "arbitrary"),
            vmem_limit_bytes=100 * 1024 * 1024),
    )(x, W_qk, W_qk)
    return out


# speedup vs baseline: 1.8256x; 1.1362x over previous
import jax
import jax.numpy as jnp
from jax.experimental import pallas as pl
from jax.experimental.pallas import tpu as pltpu

_H = 8
_NB = 5
_RB = 256
_TB = 64
_LOG2E = 1.4426950408889634


def _fused_kernel(x_ref, wq_ref, wk_ref, out_ref, stash_ref, se_ref, cm_ref):
    j = pl.program_id(1)
    N = se_ref.shape[-1]
    C = x_ref.shape[-1]
    scale = C ** -0.5

    @pl.when(j < 4)
    def _phase0():
        for t in range(2):
            h = 2 * j + t
            wk = wk_ref[t * C:(t + 1) * C, :]
            wq = wq_ref[t * C:(t + 1) * C, :]
            kt = jax.lax.dot_general(wk, x_ref[0],
                                     (((1,), (1,)), ((), ())),
                                     preferred_element_type=jnp.float32)
            for r in range(N // _RB):
                sl = pl.ds(r * _RB, _RB)
                xr = x_ref[0, sl, :]
                q = jax.lax.dot_general(xr, wq,
                                        (((1,), (1,)), ((), ())),
                                        preferred_element_type=jnp.float32)
                q = q * (scale * _LOG2E)
                logits = jax.lax.dot_general(q, kt,
                                             (((1,), (0,)), ((), ())),
                                             preferred_element_type=jnp.float32)
                e = jnp.exp2(logits)
                stash_ref[h, sl, :] = e.astype(jnp.bfloat16)
                attn = e / jnp.sum(e, axis=-1, keepdims=True)

                if t == 0:
                    @pl.when(j == 0)
                    def _():
                        se_ref[sl, :] = attn

                    @pl.when(j > 0)
                    def _():
                        se_ref[sl, :] = se_ref[sl, :] + attn
                else:
                    se_ref[sl, :] = se_ref[sl, :] + attn

        @pl.when(j == 3)
        def _():
            cm = jnp.zeros((1, N), jnp.float32)
            for r in range(N // _TB):
                se = se_ref[r * _TB:(r + 1) * _TB, :]
                col = jax.lax.broadcasted_iota(jnp.int32, (_TB, N), 1)
                for _ in range(_NB):
                    v = jnp.max(se, axis=-1, keepdims=True)
                    first = jnp.min(jnp.where(se == v, col, N),
                                    axis=-1, keepdims=True)
                    se = jnp.where(col == first, -1.0, se)
                hit = jnp.max(jnp.where(se == -1.0, 1.0, 0.0),
                              axis=0, keepdims=True)
                cm = jnp.maximum(cm, hit)
            cm_ref[...] = cm

    @pl.when(j >= 4)
    def _phase1():
        cm = cm_ref[...]
        row0 = jax.lax.broadcasted_iota(jnp.int32, (_RB, N), 0)
        col = jax.lax.broadcasted_iota(jnp.int32, (_RB, N), 1)
        for t in range(2):
            h = 2 * (j - 4) + t
            s = jnp.zeros((1, N), jnp.float32)
            for r in range(N // _RB):
                sl = pl.ds(r * _RB, _RB)
                e = stash_ref[h, sl, :].astype(jnp.float32)
                ne = e * jnp.where(col == row0 + r * _RB, 1.0, cm)
                rs = jnp.sum(ne, axis=-1, keepdims=True) + 1e-16
                nr = ne / rs
                stash_ref[h, sl, :] = nr.astype(jnp.bfloat16)
                s = s + jnp.sum(nr, axis=0, keepdims=True)
            sinv = (1.0 / (s + 1e-16)).astype(jnp.bfloat16)
            for r in range(N // _RB):
                a = stash_ref[h, r * _RB:(r + 1) * _RB, :] * sinv
                out_ref[0, t, r * _RB:(r + 1) * _RB, :] = jax.lax.dot_general(
                    a, stash_ref[h], (((1,), (1,)), ((), ())),
                    preferred_element_type=jnp.float32)


@jax.jit
def kernel(x, W_qk):
    B, N, C = x.shape
    f32 = jnp.float32
    out = pl.pallas_call(
        _fused_kernel,
        grid=(B, 8),
        in_specs=[
            pl.BlockSpec((1, N, C), lambda b, j: (b, 0, 0)),
            pl.BlockSpec((2 * C, C), lambda b, j: (jnp.minimum(j, 3), 0)),
            pl.BlockSpec((2 * C, C),
                         lambda b, j: (_H // 2 + jnp.minimum(j, 3), 0)),
        ],
        out_specs=pl.BlockSpec((1, 2, N, N),
                               lambda b, j: (b, jnp.maximum(j - 4, 0), 0, 0)),
        out_shape=jax.ShapeDtypeStruct((B, _H, N, N), f32),
        scratch_shapes=[
            pltpu.VMEM((_H, N, N), jnp.bfloat16),
            pltpu.VMEM((N, N), f32),
            pltpu.VMEM((1, N), f32),
        ],
        compiler_params=pltpu.CompilerParams(
            dimension_semantics=("arbitrary", "arbitrary"),
            vmem_limit_bytes=100 * 1024 * 1024),
    )(x, W_qk, W_qk)
    return out


# speedup vs baseline: 1.9589x; 1.0730x over previous
import jax
import jax.numpy as jnp
from jax.experimental import pallas as pl
from jax.experimental.pallas import tpu as pltpu

_H = 8
_NB = 5
_RB = 256
_TB = 64
_LOG2E = 1.4426950408889634


def _fused_kernel(x_ref, wq_ref, wk_ref, out_ref, stash_ref, se_ref, cm_ref):
    j = pl.program_id(1)
    N = se_ref.shape[-1]
    C = x_ref.shape[-1]
    scale = C ** -0.5

    @pl.when(j < 2)
    def _phase0():
        for t in range(4):
            h = 4 * j + t
            wk = wk_ref[t * C:(t + 1) * C, :]
            wq = wq_ref[t * C:(t + 1) * C, :]
            kt = jax.lax.dot_general(wk, x_ref[0],
                                     (((1,), (1,)), ((), ())),
                                     preferred_element_type=jnp.float32)
            for r in range(N // _RB):
                sl = pl.ds(r * _RB, _RB)
                xr = x_ref[0, sl, :]
                q = jax.lax.dot_general(xr, wq,
                                        (((1,), (1,)), ((), ())),
                                        preferred_element_type=jnp.float32)
                q = q * (scale * _LOG2E)
                logits = jax.lax.dot_general(q, kt,
                                             (((1,), (0,)), ((), ())),
                                             preferred_element_type=jnp.float32)
                e = jnp.exp2(logits)
                stash_ref[h, sl, :] = e.astype(jnp.bfloat16)
                attn = e / jnp.sum(e, axis=-1, keepdims=True)

                if t == 0:
                    @pl.when(j == 0)
                    def _():
                        se_ref[sl, :] = attn

                    @pl.when(j > 0)
                    def _():
                        se_ref[sl, :] = se_ref[sl, :] + attn
                else:
                    se_ref[sl, :] = se_ref[sl, :] + attn

        @pl.when(j == 1)
        def _():
            cm = jnp.zeros((1, N), jnp.float32)
            for r in range(N // _TB):
                se = se_ref[r * _TB:(r + 1) * _TB, :]
                col = jax.lax.broadcasted_iota(jnp.int32, (_TB, N), 1)
                for _ in range(_NB):
                    v = jnp.max(se, axis=-1, keepdims=True)
                    first = jnp.min(jnp.where(se == v, col, N),
                                    axis=-1, keepdims=True)
                    se = jnp.where(col == first, -1.0, se)
                hit = jnp.max(jnp.where(se == -1.0, 1.0, 0.0),
                              axis=0, keepdims=True)
                cm = jnp.maximum(cm, hit)
            cm_ref[...] = cm

    @pl.when(j >= 2)
    def _phase1():
        cm = cm_ref[...]
        row0 = jax.lax.broadcasted_iota(jnp.int32, (_RB, N), 0)
        col = jax.lax.broadcasted_iota(jnp.int32, (_RB, N), 1)
        for t in range(2):
            h = 2 * (j - 2) + t
            s = jnp.zeros((1, N), jnp.float32)
            for r in range(N // _RB):
                sl = pl.ds(r * _RB, _RB)
                e = stash_ref[h, sl, :].astype(jnp.float32)
                ne = e * jnp.where(col == row0 + r * _RB, 1.0, cm)
                rs = jnp.sum(ne, axis=-1, keepdims=True) + 1e-16
                nr = ne / rs
                stash_ref[h, sl, :] = nr.astype(jnp.bfloat16)
                s = s + jnp.sum(nr, axis=0, keepdims=True)
            sinv = (1.0 / (s + 1e-16)).astype(jnp.bfloat16)
            for r in range(N // _RB):
                a = stash_ref[h, r * _RB:(r + 1) * _RB, :] * sinv
                out_ref[0, t, r * _RB:(r + 1) * _RB, :] = jax.lax.dot_general(
                    a, stash_ref[h], (((1,), (1,)), ((), ())),
                    preferred_element_type=jnp.float32)


@jax.jit
def kernel(x, W_qk):
    B, N, C = x.shape
    f32 = jnp.float32
    out = pl.pallas_call(
        _fused_kernel,
        grid=(B, 6),
        in_specs=[
            pl.BlockSpec((1, N, C), lambda b, j: (b, 0, 0)),
            pl.BlockSpec((4 * C, C), lambda b, j: (jnp.minimum(j, 1), 0)),
            pl.BlockSpec((4 * C, C),
                         lambda b, j: (2 + jnp.minimum(j, 1), 0)),
        ],
        out_specs=pl.BlockSpec((1, 2, N, N),
                               lambda b, j: (b, jnp.maximum(j - 2, 0), 0, 0)),
        out_shape=jax.ShapeDtypeStruct((B, _H, N, N), f32),
        scratch_shapes=[
            pltpu.VMEM((_H, N, N), jnp.bfloat16),
            pltpu.VMEM((N, N), f32),
            pltpu.VMEM((1, N), f32),
        ],
        compiler_params=pltpu.CompilerParams(
            dimension_semantics=("arbitrary", "arbitrary"),
            vmem_limit_bytes=100 * 1024 * 1024),
    )(x, W_qk, W_qk)
    return out


# speedup vs baseline: 2.1911x; 1.1185x over previous
import jax
import jax.numpy as jnp
from jax.experimental import pallas as pl
from jax.experimental.pallas import tpu as pltpu

_H = 8
_NB = 5
_RB = 256
_TB = 64
_LOG2E = 1.4426950408889634


def _fused_kernel(x_ref, wq_ref, wk_ref, out_ref, stash_ref, se_ref, cm_ref):
    j = pl.program_id(1)
    N = se_ref.shape[-1]
    C = x_ref.shape[-1]
    scale = C ** -0.5

    @pl.when(j == 0)
    def _phase0():
        for t in range(_H):
            h = t
            wk = wk_ref[t * C:(t + 1) * C, :]
            wq = wq_ref[t * C:(t + 1) * C, :]
            kt = jax.lax.dot_general(wk, x_ref[0],
                                     (((1,), (1,)), ((), ())),
                                     preferred_element_type=jnp.float32)
            for r in range(N // _RB):
                sl = pl.ds(r * _RB, _RB)
                xr = x_ref[0, sl, :]
                q = jax.lax.dot_general(xr, wq,
                                        (((1,), (1,)), ((), ())),
                                        preferred_element_type=jnp.float32)
                q = q * (scale * _LOG2E)
                logits = jax.lax.dot_general(q, kt,
                                             (((1,), (0,)), ((), ())),
                                             preferred_element_type=jnp.float32)
                e = jnp.exp2(logits)
                stash_ref[h, sl, :] = e.astype(jnp.bfloat16)
                attn = e / jnp.sum(e, axis=-1, keepdims=True)

                if t == 0:
                    se_ref[sl, :] = attn
                else:
                    se_ref[sl, :] = se_ref[sl, :] + attn

        cm = jnp.zeros((1, N), jnp.float32)
        for r in range(N // _TB):
            se = se_ref[r * _TB:(r + 1) * _TB, :]
            col = jax.lax.broadcasted_iota(jnp.int32, (_TB, N), 1)
            for _ in range(_NB):
                v = jnp.max(se, axis=-1, keepdims=True)
                first = jnp.min(jnp.where(se == v, col, N),
                                axis=-1, keepdims=True)
                se = jnp.where(col == first, -1.0, se)
            hit = jnp.max(jnp.where(se == -1.0, 1.0, 0.0),
                          axis=0, keepdims=True)
            cm = jnp.maximum(cm, hit)
        cm_ref[...] = cm

    @pl.when(j >= 1)
    def _phase1():
        cm = cm_ref[...]
        row0 = jax.lax.broadcasted_iota(jnp.int32, (_RB, N), 0)
        col = jax.lax.broadcasted_iota(jnp.int32, (_RB, N), 1)
        for t in range(2):
            h = 2 * (j - 1) + t
            s = jnp.zeros((1, N), jnp.float32)
            for r in range(N // _RB):
                sl = pl.ds(r * _RB, _RB)
                e = stash_ref[h, sl, :].astype(jnp.float32)
                ne = e * jnp.where(col == row0 + r * _RB, 1.0, cm)
                rs = jnp.sum(ne, axis=-1, keepdims=True) + 1e-16
                nr = ne / rs
                stash_ref[h, sl, :] = nr.astype(jnp.bfloat16)
                s = s + jnp.sum(nr, axis=0, keepdims=True)
            sinv = (1.0 / (s + 1e-16)).astype(jnp.bfloat16)
            for r in range(N // _RB):
                a = stash_ref[h, r * _RB:(r + 1) * _RB, :] * sinv
                out_ref[0, t, r * _RB:(r + 1) * _RB, :] = jax.lax.dot_general(
                    a, stash_ref[h], (((1,), (1,)), ((), ())),
                    preferred_element_type=jnp.float32)


@jax.jit
def kernel(x, W_qk):
    B, N, C = x.shape
    f32 = jnp.float32
    out = pl.pallas_call(
        _fused_kernel,
        grid=(B, 5),
        in_specs=[
            pl.BlockSpec((1, N, C), lambda b, j: (b, 0, 0)),
            pl.BlockSpec((_H * C, C), lambda b, j: (0, 0)),
            pl.BlockSpec((_H * C, C), lambda b, j: (1, 0)),
        ],
        out_specs=pl.BlockSpec((1, 2, N, N),
                               lambda b, j: (b, jnp.maximum(j - 1, 0), 0, 0)),
        out_shape=jax.ShapeDtypeStruct((B, _H, N, N), f32),
        scratch_shapes=[
            pltpu.VMEM((_H, N, N), jnp.bfloat16),
            pltpu.VMEM((N, N), f32),
            pltpu.VMEM((1, N), f32),
        ],
        compiler_params=pltpu.CompilerParams(
            dimension_semantics=("arbitrary", "arbitrary"),
            vmem_limit_bytes=100 * 1024 * 1024),
    )(x, W_qk, W_qk)
    return out
